# baseline (speedup 1.0000x reference)
import functools

import jax
import jax.numpy as jnp
from jax import lax
from jax.experimental import pallas as pl
from jax.experimental.pallas import tpu as pltpu
from jax.experimental.pallas import tpu_sc as plsc

N_NODES = 10000
FDIM = 128
NUM_EDGES = 320000
CHUNK = 128
NUM_CORES = 2
NUM_SUBCORES = 16
NUM_TILES = NUM_CORES * NUM_SUBCORES

PAD_CHUNKS = 2560
PAD_EDGES = PAD_CHUNKS * CHUNK
TRASH_ROW = N_NODES
ACC_ROWS = N_NODES + 8

ROWS_PER_TILE = 624
ZROWS = 104

CB = PAD_CHUNKS // NUM_SUBCORES
CD = PAD_CHUNKS // NUM_TILES

_DOT_DIMS = (((1,), (1,)), ((), ()))


def _dot(a, w):
    return lax.dot_general(a, w, _DOT_DIMS, preferred_element_type=jnp.float32)


def _stage_a_body(xm, xd, ewin, w1r, w1s, w2r, w2s, b1r, b2r,
                  ym1, ym2, root1, root2, ew):
    xmb = xm[...]
    xdb = xd[...]
    ym1[...] = _dot(xmb, w1r[...])
    ym2[...] = _dot(xmb, w2r[...])
    root1[...] = _dot(xmb, w1s[...]) + b1r[...]
    root2[...] = _dot(xdb, w2s[...]) + b2r[...]
    ew[...] = jax.nn.sigmoid(ewin[...])


def _stage_c_body(agg1, root1, agg2, root2, w3r, w3s, b3r, ym3, root3):
    movie = jnp.maximum(agg1[...] + root1[...], 0.0)
    ym3[...] = _dot(movie, w3r[...])
    user = jnp.maximum(agg2[...] + root2[...], 0.0)
    root3[...] = _dot(user, w3s[...]) + b3r[...]


def _stage_e_body(agg3, root3, wl, bl, out):
    a3 = agg3[...]
    user = jnp.maximum(a3[0] + a3[1] + root3[...], 0.0)
    out[...] = _dot(user, wl[...]) + bl[...]


_GRID = 10
_ROWB = N_NODES // _GRID

_node_spec = pl.BlockSpec((_ROWB, FDIM), lambda i: (i, 0))
_w_spec = pl.BlockSpec((FDIM, FDIM), lambda i: (0, 0))
_b_spec = pl.BlockSpec((1, FDIM), lambda i: (0, 0))
_ew_spec = pl.BlockSpec((PAD_CHUNKS // _GRID, FDIM), lambda i: (i, 0))
_node_sds = jax.ShapeDtypeStruct((N_NODES, FDIM), jnp.float32)


def _stage_a(xm, xd, ew2d, w1r, w1s, w2r, w2s, b1r, b2r):
    return pl.pallas_call(
        _stage_a_body,
        grid=(_GRID,),
        in_specs=[_node_spec, _node_spec, _ew_spec,
                  _w_spec, _w_spec, _w_spec, _w_spec, _b_spec, _b_spec],
        out_specs=[_node_spec, _node_spec, _node_spec, _node_spec, _ew_spec],
        out_shape=[_node_sds, _node_sds, _node_sds, _node_sds,
                   jax.ShapeDtypeStruct((PAD_CHUNKS, FDIM), jnp.float32)],
    )(xm, xd, ew2d, w1r, w1s, w2r, w2s, b1r, b2r)


def _stage_c(agg1, root1, agg2, root2, w3r, w3s, b3r):
    return pl.pallas_call(
        _stage_c_body,
        grid=(_GRID,),
        in_specs=[_node_spec, _node_spec, _node_spec, _node_spec,
                  _w_spec, _w_spec, _b_spec],
        out_specs=[_node_spec, _node_spec],
        out_shape=[_node_sds, _node_sds],
    )(agg1, root1, agg2, root2, w3r, w3s, b3r)


def _stage_e(agg3, root3, wl, bl):
    return pl.pallas_call(
        _stage_e_body,
        grid=(_GRID,),
        in_specs=[pl.BlockSpec((2, _ROWB, FDIM), lambda i: (0, i, 0)),
                  _node_spec, _w_spec, _b_spec],
        out_specs=_node_spec,
        out_shape=_node_sds,
    )(agg3, root3, wl, bl)


BATCH = 16
PAIRS = BATCH // 2


def _zero_acc(zbuf, acc, s):
    def zrow(i, carry):
        for k in range(FDIM // 16):
            zbuf[i, pl.ds(16 * k, 16)] = jnp.zeros((16,), jnp.float32)
        return carry
    lax.fori_loop(0, CHUNK, zrow, 0)
    base = s * ROWS_PER_TILE
    for j in range(4):
        pltpu.sync_copy(zbuf, acc.at[pl.ds(base + j * CHUNK, CHUNK)])
    pltpu.sync_copy(zbuf.at[pl.ds(0, ROWS_PER_TILE - 4 * CHUNK)],
                    acc.at[pl.ds(base + 4 * CHUNK, ROWS_PER_TILE - 4 * CHUNK)])

    @pl.when(s == NUM_SUBCORES - 1)
    def _():
        tail = ACC_ROWS - NUM_SUBCORES * ROWS_PER_TILE
        pltpu.sync_copy(zbuf.at[pl.ds(0, tail)],
                        acc.at[pl.ds(NUM_SUBCORES * ROWS_PER_TILE, tail)])


def _scale_rows(rows, wS, t):
    def sc16(j, carry):
        wv = wS[t, pl.ds(j * 16, 16)]
        for l in range(16):
            wvec = lax.full((16,), wv[l], jnp.float32)
            r = j * 16 + l
            for k in range(FDIM // 16):
                rows[r, pl.ds(16 * k, 16)] = rows[r, pl.ds(16 * k, 16)] * wvec
        return carry
    lax.fori_loop(0, CHUNK // 16, sc16, 0)


def _run_conv(tab, src2, dst2, w2, weighted, out, scratches, s, start, nb):
    srcb, dstb, wb, rows0, rows1, acc, g0, g1, s0, s1, isem = scratches
    slots = [(srcb.at[0], dstb.at[0], wb.at[0]),
             (srcb.at[1], dstb.at[1], wb.at[1])]

    def load_idx_async(slot, b):
        srcS, dstS, wS = slots[slot]
        off = start + b * BATCH
        pltpu.async_copy(src2.at[pl.ds(off, BATCH)], srcS, isem)
        pltpu.async_copy(dst2.at[pl.ds(off, BATCH)], dstS, isem)
        if weighted:
            pltpu.async_copy(w2.at[pl.ds(off, BATCH)], wS, isem)

    def wait_idx(slot, b):
        srcS, dstS, wS = slots[slot]
        off = start + b * BATCH
        pltpu.make_async_copy(src2.at[pl.ds(off, BATCH)], srcS, isem).wait()
        pltpu.make_async_copy(dst2.at[pl.ds(off, BATCH)], dstS, isem).wait()
        if weighted:
            pltpu.make_async_copy(w2.at[pl.ds(off, BATCH)], wS, isem).wait()

    def batch_body(slot, b):
        srcS, dstS, wS = slots[slot]

        @pl.when(b > 0)
        def _():
            wait_idx(slot, b)

        @pl.when(b + 1 < nb)
        def _():
            load_idx_async(1 - slot, b + 1)
        pltpu.async_copy(tab.at[srcS.at[0]], rows0, g0)

        def pair(u, carry):
            la = 2 * u
            lb = la + 1
            pltpu.make_async_copy(tab.at[srcS.at[la]], rows0, g0).wait()

            @pl.when(u > 0)
            def _():
                pltpu.make_async_copy(rows1, acc.at[dstS.at[la - 1]], s1).wait()
            pltpu.async_copy(tab.at[srcS.at[lb]], rows1, g1)
            if weighted:
                _scale_rows(rows0, wS, la)
            pltpu.async_copy(rows0, acc.at[dstS.at[la]], s0, add=True)

            pltpu.make_async_copy(tab.at[srcS.at[lb]], rows1, g1).wait()

            @pl.when(u + 1 < PAIRS)
            def _():
                pltpu.make_async_copy(rows0, acc.at[dstS.at[la]], s0).wait()
                pltpu.async_copy(tab.at[srcS.at[la + 2]], rows0, g0)
            if weighted:
                _scale_rows(rows1, wS, lb)
            pltpu.async_copy(rows1, acc.at[dstS.at[lb]], s1, add=True)
            return carry
        lax.fori_loop(0, PAIRS, pair, 0)
        pltpu.make_async_copy(rows0, acc.at[dstS.at[BATCH - 2]], s0).wait()
        pltpu.make_async_copy(rows1, acc.at[dstS.at[BATCH - 1]], s1).wait()

    srcS0, dstS0, wS0 = slots[0]
    off0 = start
    pltpu.sync_copy(src2.at[pl.ds(off0, BATCH)], srcS0)
    pltpu.sync_copy(dst2.at[pl.ds(off0, BATCH)], dstS0)
    if weighted:
        pltpu.sync_copy(w2.at[pl.ds(off0, BATCH)], wS0)
    _zero_acc(rows1, acc, s)
    plsc.subcore_barrier()

    def bloop(b, carry):
        parity = lax.rem(b, 2)

        @pl.when(parity == 0)
        def _():
            batch_body(0, b)

        @pl.when(parity == 1)
        def _():
            batch_body(1, b)
        return carry
    lax.fori_loop(0, nb, bloop, 0)
    plsc.subcore_barrier()

    pltpu.sync_copy(acc.at[pl.ds(s * ROWS_PER_TILE, ROWS_PER_TILE)],
                    out.at[pl.ds(s * ROWS_PER_TILE, ROWS_PER_TILE)])

    @pl.when(s == NUM_SUBCORES - 1)
    def _():
        tail = N_NODES - NUM_SUBCORES * ROWS_PER_TILE
        base = NUM_SUBCORES * ROWS_PER_TILE
        pltpu.sync_copy(acc.at[pl.ds(base, tail)], out.at[pl.ds(base, tail)])


def _sc_scratch():
    return [
        pltpu.VMEM((2, BATCH, CHUNK), jnp.int32),
        pltpu.VMEM((2, BATCH, CHUNK), jnp.int32),
        pltpu.VMEM((2, BATCH, CHUNK), jnp.float32),
        pltpu.VMEM((CHUNK, FDIM), jnp.float32),
        pltpu.VMEM((CHUNK, FDIM), jnp.float32),
        pltpu.VMEM_SHARED((ACC_ROWS, FDIM), jnp.float32),
        pltpu.SemaphoreType.DMA,
        pltpu.SemaphoreType.DMA,
        pltpu.SemaphoreType.DMA,
        pltpu.SemaphoreType.DMA,
        pltpu.SemaphoreType.DMA,
    ]


@functools.lru_cache(maxsize=None)
def _build_sc_kernels():
    mesh = plsc.VectorSubcoreMesh(core_axis_name="c", subcore_axis_name="s")

    @functools.partial(
        pl.kernel, mesh=mesh,
        out_type=[_node_sds, _node_sds],
        scratch_types=_sc_scratch(),
    )
    def sc_stage_b(tab1, src1, dst1, tab2, src2, dst2, ew,
                   out1, out2, *scratches):
        c = lax.axis_index("c")
        s = lax.axis_index("s")

        @pl.when(c == 0)
        def _():
            _run_conv(tab1, src1, dst1, None, False, out1, scratches,
                      s, s * CB, CB // BATCH)

        @pl.when(c == 1)
        def _():
            _run_conv(tab2, src2, dst2, ew, True, out2, scratches,
                      s, s * CB, CB // BATCH)

    @functools.partial(
        pl.kernel, mesh=mesh,
        out_type=jax.ShapeDtypeStruct((2, N_NODES, FDIM), jnp.float32),
        scratch_types=_sc_scratch(),
    )
    def sc_stage_d(tab, src, dst, ew, out, *scratches):
        c = lax.axis_index("c")
        s = lax.axis_index("s")
        w = s * NUM_CORES + c
        _run_conv(tab, src, dst, ew, True, out.at[c], scratches,
                  s, w * CD, CD // BATCH)

    return sc_stage_b, sc_stage_d


def _pad_idx(a, fill):
    pad = jnp.full((PAD_EDGES - NUM_EDGES,), fill, a.dtype)
    return jnp.concatenate([a, pad]).reshape(PAD_CHUNKS, CHUNK)


def kernel(x_measurement, x_demand, edge_index_mm, edge_index_md, edge_weight,
           W1r, b1r, W1s, W2r, b2r, W2s, W3r, b3r, W3s, Wl, bl):
    src_mm = _pad_idx(edge_index_mm[0], 0)
    dst_mm = _pad_idx(edge_index_mm[1], TRASH_ROW)
    src_md = _pad_idx(edge_index_md[0], 0)
    dst_md = _pad_idx(edge_index_md[1], TRASH_ROW)
    ew2d = _pad_idx(edge_weight, 0.0)

    ym1, ym2, root1, root2, ew2d = _stage_a(
        x_measurement, x_demand, ew2d, W1r, W1s, W2r, W2s,
        b1r.reshape(1, FDIM), b2r.reshape(1, FDIM))

    sc_stage_b, sc_stage_d = _build_sc_kernels()
    agg1, agg2 = sc_stage_b(ym1, src_mm, dst_mm, ym2, src_md, dst_md, ew2d)

    ym3, root3 = _stage_c(agg1, root1, agg2, root2, W3r, W3s,
                          b3r.reshape(1, FDIM))

    agg3 = sc_stage_d(ym3, src_md, dst_md, ew2d)

    return _stage_e(agg3, root3, Wl, bl.reshape(1, FDIM))


# speedup vs baseline: 1.2035x; 1.2035x over previous
import functools

import jax
import jax.numpy as jnp
from jax import lax
from jax.experimental import pallas as pl
from jax.experimental.pallas import tpu as pltpu
from jax.experimental.pallas import tpu_sc as plsc

N_NODES = 10000
FDIM = 128
NUM_EDGES = 320000
NUM_CORES = 2
NUM_SUBCORES = 16
NUM_TILES = NUM_CORES * NUM_SUBCORES

CHUNK = 80
SLOTS = 4
BATCH = 8
PAD_CHUNKS = 4096
PAD_EDGES = PAD_CHUNKS * CHUNK
ZERO_ROW = N_NODES

ROWS_PER_TILE = 624

CB = PAD_CHUNKS // NUM_SUBCORES
CD = PAD_CHUNKS // NUM_TILES

_DOT_DIMS = (((1,), (1,)), ((), ()))


def _dot(a, w):
    return lax.dot_general(a, w, _DOT_DIMS, preferred_element_type=jnp.float32)


def _stage_a_body(xm, xd, ewin, w1r, w1s, w2r, w2s, b1r, b2r,
                  ym1, ym2, root1, root2, ew):
    xmb = xm[...]
    xdb = xd[...]
    ym1[...] = _dot(xmb, w1r[...])
    ym2[...] = _dot(xmb, w2r[...])
    root1[...] = _dot(xmb, w1s[...]) + b1r[...]
    root2[...] = _dot(xdb, w2s[...]) + b2r[...]
    ew[...] = jax.nn.sigmoid(ewin[...])


def _stage_c_body(agg1, root1, agg2, root2, w3r, w3s, b3r, ym3, root3):
    movie = jnp.maximum(agg1[...] + root1[...], 0.0)
    ym3[...] = _dot(movie, w3r[...])
    user = jnp.maximum(agg2[...] + root2[...], 0.0)
    root3[...] = _dot(user, w3s[...]) + b3r[...]


def _stage_e_body(agg3, root3, wl, bl, out):
    a3 = agg3[...]
    user = jnp.maximum(a3[0] + a3[1] + root3[...], 0.0)
    out[...] = _dot(user, wl[...]) + bl[...]


_GRID = 10
_ROWB = N_NODES // _GRID
_EWROWS = PAD_EDGES // FDIM

_node_spec = pl.BlockSpec((_ROWB, FDIM), lambda i: (i, 0))
_w_spec = pl.BlockSpec((FDIM, FDIM), lambda i: (0, 0))
_b_spec = pl.BlockSpec((1, FDIM), lambda i: (0, 0))
_ew_spec = pl.BlockSpec((_EWROWS // _GRID, FDIM), lambda i: (i, 0))
_node_sds = jax.ShapeDtypeStruct((N_NODES, FDIM), jnp.float32)


def _stage_a(xm, xd, ew2d, w1r, w1s, w2r, w2s, b1r, b2r):
    return pl.pallas_call(
        _stage_a_body,
        grid=(_GRID,),
        in_specs=[_node_spec, _node_spec, _ew_spec,
                  _w_spec, _w_spec, _w_spec, _w_spec, _b_spec, _b_spec],
        out_specs=[_node_spec, _node_spec, _node_spec, _node_spec, _ew_spec],
        out_shape=[_node_sds, _node_sds, _node_sds, _node_sds,
                   jax.ShapeDtypeStruct((_EWROWS, FDIM), jnp.float32)],
    )(xm, xd, ew2d, w1r, w1s, w2r, w2s, b1r, b2r)


def _stage_c(agg1, root1, agg2, root2, w3r, w3s, b3r):
    return pl.pallas_call(
        _stage_c_body,
        grid=(_GRID,),
        in_specs=[_node_spec, _node_spec, _node_spec, _node_spec,
                  _w_spec, _w_spec, _b_spec],
        out_specs=[_node_spec, _node_spec],
        out_shape=[_node_sds, _node_sds],
    )(agg1, root1, agg2, root2, w3r, w3s, b3r)


def _stage_e(agg3, root3, wl, bl):
    return pl.pallas_call(
        _stage_e_body,
        grid=(_GRID,),
        in_specs=[pl.BlockSpec((2, _ROWB, FDIM), lambda i: (0, i, 0)),
                  _node_spec, _w_spec, _b_spec],
        out_specs=_node_spec,
        out_shape=_node_sds,
    )(agg3, root3, wl, bl)


def _zero_acc(zbuf, acc, s):
    def zrow(i, carry):
        for k in range(FDIM // 16):
            zbuf[i, pl.ds(16 * k, 16)] = jnp.zeros((16,), jnp.float32)
        return carry
    lax.fori_loop(0, CHUNK, zrow, 0)
    base = s * ROWS_PER_TILE
    nfull = ROWS_PER_TILE // CHUNK
    for j in range(nfull):
        pltpu.sync_copy(zbuf, acc.at[pl.ds(base + j * CHUNK, CHUNK)])
    rem = ROWS_PER_TILE - nfull * CHUNK
    pltpu.sync_copy(zbuf.at[pl.ds(0, rem)],
                    acc.at[pl.ds(base + nfull * CHUNK, rem)])

    @pl.when(s == NUM_SUBCORES - 1)
    def _():
        tail = N_NODES - NUM_SUBCORES * ROWS_PER_TILE
        pltpu.sync_copy(zbuf.at[pl.ds(0, tail)],
                        acc.at[pl.ds(NUM_SUBCORES * ROWS_PER_TILE, tail)])


def _scale_rows(rows, wb, trow):
    def sc16(j, carry):
        wv = wb[trow, pl.ds(j * 16, 16)]
        for l in range(16):
            wvec = lax.full((16,), wv[l], jnp.float32)
            r = j * 16 + l
            for k in range(FDIM // 16):
                rows[r, pl.ds(16 * k, 16)] = rows[r, pl.ds(16 * k, 16)] * wvec
        return carry
    lax.fori_loop(0, CHUNK // 16, sc16, 0)


def _run_conv(tab, src2, dst2, w2, weighted, out, scratches, s, start, nb):
    (srcb, dstb, wb, r0, r1, r2, r3, acc,
     g0, g1, g2, g3, s0, s1, s2, s3, isem) = scratches
    rows = [r0, r1, r2, r3]
    gsem = [g0, g1, g2, g3]
    ssem = [s0, s1, s2, s3]

    def idx_copies(b, ibase):
        off = start + b * BATCH
        return [(src2.at[pl.ds(off, BATCH)], srcb.at[pl.ds(ibase, BATCH)]),
                (dst2.at[pl.ds(off, BATCH)], dstb.at[pl.ds(ibase, BATCH)])] + (
                [(w2.at[pl.ds(off, BATCH)], wb.at[pl.ds(ibase, BATCH)])]
                if weighted else [])

    for a, bdst in idx_copies(0, 0):
        pltpu.sync_copy(a, bdst)
    if nb > 1:
        for a, bdst in idx_copies(1, BATCH):
            pltpu.async_copy(a, bdst, isem)
    _zero_acc(rows[3], acc, s)
    pltpu.async_copy(tab.at[srcb.at[0]], rows[0], gsem[0])
    pltpu.async_copy(tab.at[srcb.at[1]], rows[1], gsem[1])
    pltpu.async_copy(tab.at[srcb.at[2]], rows[2], gsem[2])
    plsc.subcore_barrier()

    def batch_iter(b, carry):
        ibase = lax.rem(b, 2) * BATCH
        nibase = lax.rem(b + 1, 2) * BATCH
        pbase = lax.rem(b + 1, 2) * BATCH

        @pl.when(b + 1 < nb)
        def _():
            for a, bdst in idx_copies(b + 1, nibase):
                pltpu.async_copy(a, bdst, isem)

        for j in range(BATCH):
            trow = ibase + j
            p = j % SLOTS
            r = (j + 3) % SLOTS
            pltpu.make_async_copy(tab.at[srcb.at[trow]],
                                  rows[p], gsem[p]).wait()
            if j == 0:
                @pl.when(b > 0)
                def _():
                    pltpu.make_async_copy(
                        rows[r], acc.at[dstb.at[pbase + BATCH - 1]],
                        ssem[r]).wait()
            else:
                pltpu.make_async_copy(
                    rows[r], acc.at[dstb.at[ibase + j - 1]], ssem[r]).wait()
            if j == 2:
                @pl.when(b + 1 < nb)
                def _():
                    for a, bdst in idx_copies(b + 1, nibase):
                        pltpu.make_async_copy(a, bdst, isem).wait()
            if j < BATCH - 3:
                pltpu.async_copy(tab.at[srcb.at[ibase + j + 3]],
                                 rows[r], gsem[r])
            else:
                @pl.when(b + 1 < nb)
                def _():
                    pltpu.async_copy(tab.at[srcb.at[nibase + j + 3 - BATCH]],
                                     rows[r], gsem[r])
            if weighted:
                _scale_rows(rows[p], wb, trow)
            pltpu.async_copy(rows[p], acc.at[dstb.at[trow]],
                             ssem[p], add=True)
        return carry
    lax.fori_loop(0, nb, batch_iter, 0)

    lbase = ((nb - 1) % 2) * BATCH
    pltpu.make_async_copy(rows[(BATCH - 1) % SLOTS],
                          acc.at[dstb.at[lbase + BATCH - 1]],
                          ssem[(BATCH - 1) % SLOTS]).wait()
    plsc.subcore_barrier()

    pltpu.sync_copy(acc.at[pl.ds(s * ROWS_PER_TILE, ROWS_PER_TILE)],
                    out.at[pl.ds(s * ROWS_PER_TILE, ROWS_PER_TILE)])

    @pl.when(s == NUM_SUBCORES - 1)
    def _():
        tail = N_NODES - NUM_SUBCORES * ROWS_PER_TILE
        base = NUM_SUBCORES * ROWS_PER_TILE
        pltpu.sync_copy(acc.at[pl.ds(base, tail)], out.at[pl.ds(base, tail)])


def _sc_scratch():
    return ([
        pltpu.VMEM((2 * BATCH, CHUNK), jnp.int32),
        pltpu.VMEM((2 * BATCH, CHUNK), jnp.int32),
        pltpu.VMEM((2 * BATCH, CHUNK), jnp.float32),
    ] + [pltpu.VMEM((CHUNK, FDIM), jnp.float32) for _ in range(SLOTS)]
      + [pltpu.VMEM_SHARED((N_NODES, FDIM), jnp.float32)]
      + [pltpu.SemaphoreType.DMA for _ in range(2 * SLOTS + 1)])


@functools.lru_cache(maxsize=None)
def _build_sc_kernels():
    mesh = plsc.VectorSubcoreMesh(core_axis_name="c", subcore_axis_name="s")

    @functools.partial(
        pl.kernel, mesh=mesh,
        out_type=[_node_sds, _node_sds],
        scratch_types=_sc_scratch(),
    )
    def sc_stage_b(tab1, src1, dst1, tab2, src2, dst2, ew,
                   out1, out2, *scratches):
        c = lax.axis_index("c")
        s = lax.axis_index("s")

        @pl.when(c == 0)
        def _():
            _run_conv(tab1, src1, dst1, None, False, out1, scratches,
                      s, s * CB, CB // BATCH)

        @pl.when(c == 1)
        def _():
            _run_conv(tab2, src2, dst2, ew, True, out2, scratches,
                      s, s * CB, CB // BATCH)

    @functools.partial(
        pl.kernel, mesh=mesh,
        out_type=jax.ShapeDtypeStruct((2, N_NODES, FDIM), jnp.float32),
        scratch_types=_sc_scratch(),
    )
    def sc_stage_d(tab, src, dst, ew, out, *scratches):
        c = lax.axis_index("c")
        s = lax.axis_index("s")
        w = s * NUM_CORES + c
        _run_conv(tab, src, dst, ew, True, out.at[c], scratches,
                  s, w * CD, CD // BATCH)

    return sc_stage_b, sc_stage_d


def _pad_idx(a, fill):
    pad = jnp.full((PAD_EDGES - NUM_EDGES,), fill, a.dtype)
    return jnp.concatenate([a, pad]).reshape(PAD_CHUNKS, CHUNK)


def kernel(x_measurement, x_demand, edge_index_mm, edge_index_md, edge_weight,
           W1r, b1r, W1s, W2r, b2r, W2s, W3r, b3r, W3s, Wl, bl):
    src_mm = _pad_idx(edge_index_mm[0], ZERO_ROW)
    dst_mm = _pad_idx(edge_index_mm[1], 0)
    src_md = _pad_idx(edge_index_md[0], 0)
    dst_md = _pad_idx(edge_index_md[1], 0)
    ew2d = _pad_idx(edge_weight, float("-inf")).reshape(_EWROWS, FDIM)

    ym1, ym2, root1, root2, ew2d = _stage_a(
        x_measurement, x_demand, ew2d, W1r, W1s, W2r, W2s,
        b1r.reshape(1, FDIM), b2r.reshape(1, FDIM))
    ew = ew2d.reshape(PAD_CHUNKS, CHUNK)

    ym1p = jnp.concatenate(
        [ym1, jnp.zeros((8, FDIM), jnp.float32)], axis=0)

    sc_stage_b, sc_stage_d = _build_sc_kernels()
    agg1, agg2 = sc_stage_b(ym1p, src_mm, dst_mm, ym2, src_md, dst_md, ew)

    ym3, root3 = _stage_c(agg1, root1, agg2, root2, W3r, W3s,
                          b3r.reshape(1, FDIM))

    agg3 = sc_stage_d(ym3, src_md, dst_md, ew)

    return _stage_e(agg3, root3, Wl, bl.reshape(1, FDIM))
